# baseline (speedup 1.0000x reference)
import functools

import jax
import jax.numpy as jnp
from jax import lax
from jax.experimental import pallas as pl
from jax.experimental.pallas import tpu as pltpu
from jax.experimental.pallas import tpu_sc as plsc

_SDF_MIN = -7.0
_SDF_MAX = 7.0
_N_BINS = 256

_NC = 2
_NS = 16
_NW = _NC * _NS
_L = 16

_N = 8 * 128 * 128 * 128
_PER_W = _N // _NW
_CHUNK = 16384
_NCHUNKS = _PER_W // _CHUNK
_UNROLL = 8

_A = (_N_BINS - 1) / (_SDF_MAX - _SDF_MIN)
_B = -_SDF_MIN * _A + 0.5


def _compute_chunk(yp_v, yt_v, lutrep_v, lane, accs):
    def body(i, acc_in):
        off = i * (_UNROLL * _L)
        acc = list(acc_in)
        for j in range(_UNROLL):
            t = yt_v[pl.ds(off + j * _L, _L)]
            p = yp_v[pl.ds(off + j * _L, _L)]
            tc = jnp.minimum(jnp.maximum(t, _SDF_MIN), _SDF_MAX)
            x = tc * _A + _B
            idx = x.astype(jnp.int32)
            w = plsc.load_gather(lutrep_v, [(idx << 4) + lane])
            d = p - t
            acc[j] = acc[j] + w * (d * d)
        return tuple(acc)

    return lax.fori_loop(0, _CHUNK // (_UNROLL * _L), body, accs)


def _sc_body(yp_hbm, yt_hbm, lut_hbm, out_hbm,
             lut_v, lutrep_v, yp0, yp1, yt0, yt1, acc_v,
             sp0, sp1, st0, st1):
    c = lax.axis_index("c")
    s = lax.axis_index("s")
    wid = s * _NC + c
    base = wid * _PER_W
    pltpu.sync_copy(lut_hbm, lut_v)
    lane = lax.iota(jnp.int32, _L)

    def rep_body(g, carry):
        wv = lut_v[pl.ds(g * _L, _L)]
        for k in range(_L):
            lutrep_v[pl.ds((g * _L + k) * _L, _L)] = jnp.full(
                (_L,), wv[k], jnp.float32)
        return carry

    lax.fori_loop(0, _N_BINS // _L, rep_body, jnp.int32(0))

    bufs = ((yp0, yt0, sp0, st0), (yp1, yt1, sp1, st1))

    def start(k, parity):
        ypb, ytb, sp, st = bufs[parity]
        off = base + k * _CHUNK
        pltpu.async_copy(yp_hbm.at[pl.ds(off, _CHUNK)], ypb, sp)
        pltpu.async_copy(yt_hbm.at[pl.ds(off, _CHUNK)], ytb, st)

    def wait(parity):
        ypb, ytb, sp, st = bufs[parity]
        pltpu.make_async_copy(yp_hbm.at[pl.ds(base, _CHUNK)], ypb, sp).wait()
        pltpu.make_async_copy(yt_hbm.at[pl.ds(base, _CHUNK)], ytb, st).wait()

    start(0, 0)
    start(1, 1)

    zeros = jnp.zeros((_L,), jnp.float32)
    accs0 = (zeros,) * _UNROLL

    def pair_body(g, accs):
        k0 = 2 * g
        wait(0)
        accs = _compute_chunk(bufs[0][0], bufs[0][1], lutrep_v, lane, accs)
        start(k0 + 2, 0)
        wait(1)
        accs = _compute_chunk(bufs[1][0], bufs[1][1], lutrep_v, lane, accs)
        start(k0 + 3, 1)
        return accs

    accs = lax.fori_loop(0, _NCHUNKS // 2 - 1, pair_body, accs0)

    wait(0)
    accs = _compute_chunk(bufs[0][0], bufs[0][1], lutrep_v, lane, accs)
    wait(1)
    accs = _compute_chunk(bufs[1][0], bufs[1][1], lutrep_v, lane, accs)

    half = len(accs) // 2
    acc = sum(accs[1:half], accs[0]) + sum(accs[half + 1:], accs[half])
    acc_v[...] = acc
    pltpu.sync_copy(acc_v, out_hbm.at[wid])


@jax.jit
def _sc_partials(yp, yt, lut):
    mesh = plsc.VectorSubcoreMesh(core_axis_name="c", subcore_axis_name="s")
    return pl.kernel(
        _sc_body,
        out_type=jax.ShapeDtypeStruct((_NW, _L), jnp.float32),
        mesh=mesh,
        scratch_types=[
            pltpu.VMEM((_N_BINS,), jnp.float32),
            pltpu.VMEM((_N_BINS * _L,), jnp.float32),
            pltpu.VMEM((_CHUNK,), jnp.float32),
            pltpu.VMEM((_CHUNK,), jnp.float32),
            pltpu.VMEM((_CHUNK,), jnp.float32),
            pltpu.VMEM((_CHUNK,), jnp.float32),
            pltpu.VMEM((_L,), jnp.float32),
            pltpu.SemaphoreType.DMA,
            pltpu.SemaphoreType.DMA,
            pltpu.SemaphoreType.DMA,
            pltpu.SemaphoreType.DMA,
        ],
        compiler_params=pltpu.CompilerParams(needs_layout_passes=False),
    )(yp, yt, lut)


def kernel(y_pred, y_true, lut):
    n = y_pred.size
    partials = _sc_partials(y_pred.reshape(-1), y_true.reshape(-1), lut)
    return (partials.sum() / n).astype(jnp.float32)


# speedup vs baseline: 1.1625x; 1.1625x over previous
import jax
import jax.numpy as jnp
from jax import lax
from jax.experimental import pallas as pl
from jax.experimental.pallas import tpu as pltpu
from jax.experimental.pallas import tpu_sc as plsc

_SDF_MIN = -7.0
_SDF_MAX = 7.0
_N_BINS = 256

_NC = 2
_NS = 16
_NW = _NC * _NS
_L = 16

_N = 8 * 128 * 128 * 128
_PER_W = _N // _NW
_CHUNK = 16384
_NCHUNKS = _PER_W // _CHUNK
_UNROLL = 8

_A = (_N_BINS - 1) / (_SDF_MAX - _SDF_MIN)
_B = -_SDF_MIN * _A + 0.5


def _compute_chunk(yp_v, yt_v, lut_v, accs):
    def body(i, acc_in):
        off = i * (_UNROLL * _L)
        acc = list(acc_in)
        for j in range(_UNROLL):
            t = yt_v[pl.ds(off + j * _L, _L)]
            p = yp_v[pl.ds(off + j * _L, _L)]
            tc = jnp.minimum(jnp.maximum(t, _SDF_MIN), _SDF_MAX)
            x = tc * _A + _B
            idx = x.astype(jnp.int32)
            w = plsc.load_gather(lut_v, [idx])
            d = p - t
            acc[j] = acc[j] + w * (d * d)
        return tuple(acc)

    return lax.fori_loop(0, _CHUNK // (_UNROLL * _L), body, accs)


def _sc_body(yp_hbm, yt_hbm, lut_hbm, out_hbm,
             lut_v, yp0, yp1, yt0, yt1, acc_v,
             sp0, sp1, st0, st1):
    c = lax.axis_index("c")
    s = lax.axis_index("s")
    wid = s * _NC + c
    base = wid * _PER_W
    pltpu.sync_copy(lut_hbm, lut_v)

    bufs = ((yp0, yt0, sp0, st0), (yp1, yt1, sp1, st1))

    def start(k, parity):
        ypb, ytb, sp, st = bufs[parity]
        off = base + k * _CHUNK
        pltpu.async_copy(yp_hbm.at[pl.ds(off, _CHUNK)], ypb, sp)
        pltpu.async_copy(yt_hbm.at[pl.ds(off, _CHUNK)], ytb, st)

    def wait(parity):
        ypb, ytb, sp, st = bufs[parity]
        pltpu.make_async_copy(yp_hbm.at[pl.ds(base, _CHUNK)], ypb, sp).wait()
        pltpu.make_async_copy(yt_hbm.at[pl.ds(base, _CHUNK)], ytb, st).wait()

    start(0, 0)
    start(1, 1)

    zeros = jnp.zeros((_L,), jnp.float32)
    accs0 = (zeros,) * _UNROLL

    def pair_body(g, accs):
        k0 = 2 * g
        wait(0)
        accs = _compute_chunk(bufs[0][0], bufs[0][1], lut_v, accs)
        start(k0 + 2, 0)
        wait(1)
        accs = _compute_chunk(bufs[1][0], bufs[1][1], lut_v, accs)
        start(k0 + 3, 1)
        return accs

    accs = lax.fori_loop(0, _NCHUNKS // 2 - 1, pair_body, accs0)

    wait(0)
    accs = _compute_chunk(bufs[0][0], bufs[0][1], lut_v, accs)
    wait(1)
    accs = _compute_chunk(bufs[1][0], bufs[1][1], lut_v, accs)

    half = len(accs) // 2
    acc = sum(accs[1:half], accs[0]) + sum(accs[half + 1:], accs[half])
    acc_v[...] = acc
    pltpu.sync_copy(acc_v, out_hbm.at[wid])


@jax.jit
def _sc_partials(yp, yt, lut):
    mesh = plsc.VectorSubcoreMesh(core_axis_name="c", subcore_axis_name="s")
    return pl.kernel(
        _sc_body,
        out_type=jax.ShapeDtypeStruct((_NW, _L), jnp.float32),
        mesh=mesh,
        scratch_types=[
            pltpu.VMEM((_N_BINS,), jnp.float32),
            pltpu.VMEM((_CHUNK,), jnp.float32),
            pltpu.VMEM((_CHUNK,), jnp.float32),
            pltpu.VMEM((_CHUNK,), jnp.float32),
            pltpu.VMEM((_CHUNK,), jnp.float32),
            pltpu.VMEM((_L,), jnp.float32),
            pltpu.SemaphoreType.DMA,
            pltpu.SemaphoreType.DMA,
            pltpu.SemaphoreType.DMA,
            pltpu.SemaphoreType.DMA,
        ],
        compiler_params=pltpu.CompilerParams(needs_layout_passes=False),
    )(yp, yt, lut)


def kernel(y_pred, y_true, lut):
    n = y_pred.size
    partials = _sc_partials(y_pred.reshape(-1), y_true.reshape(-1), lut)
    return (partials.sum() / n).astype(jnp.float32)
